# baseline (speedup 1.0000x reference)
import functools

import numpy as np
import jax
import jax.numpy as jnp
from jax import lax
from jax.experimental import pallas as pl
from jax.experimental.pallas import tpu as pltpu
from jax.experimental.pallas import tpu_sc as plsc

D_MODEL = 64
SEQ_LEN = 200
BATCH = 1024
SCALE = np.float32(np.sqrt(D_MODEL))


def _positional_encoding(length, depth):
    half = depth / 2
    positions = np.arange(length)[:, np.newaxis]
    depths = np.arange(half)[np.newaxis, :] / half
    angle_rates = 1 / 10000 ** depths
    angle_rads = positions * angle_rates
    pe = np.concatenate([np.sin(angle_rads), np.cos(angle_rads)], axis=-1)
    return pe.astype(np.float32)


_PE_NP = _positional_encoding(SEQ_LEN, D_MODEL)

_SPLIT_A = 128
_SPLIT_B = SEQ_LEN - _SPLIT_A


@functools.cache
def _build_emb_lookup():
    info = plsc.get_sparse_core_info()
    nc, ns = info.num_cores, info.num_subcores
    nw = nc * ns
    seq_per_w = BATCH // nw
    mesh = plsc.VectorSubcoreMesh(core_axis_name="c", subcore_axis_name="s")

    @functools.partial(
        pl.kernel,
        mesh=mesh,
        out_type=jax.ShapeDtypeStruct((BATCH * SEQ_LEN, D_MODEL), jnp.float32),
        scratch_types=[
            pltpu.VMEM((_SPLIT_A,), jnp.int32),
            pltpu.VMEM((_SPLIT_B,), jnp.int32),
            pltpu.VMEM((SEQ_LEN, D_MODEL), jnp.float32),
            pltpu.VMEM((SEQ_LEN, D_MODEL), jnp.float32),
            pltpu.SemaphoreType.DMA,
        ],
        compiler_params=pltpu.CompilerParams(use_tc_tiling_on_sc=False),
    )
    def _emb_lookup(idx_hbm, table_hbm, pe_hbm, out_hbm,
                    idx_a, idx_b, buf_v, pe_v, sem):
        wid = lax.axis_index("s") * nc + lax.axis_index("c")

        pltpu.sync_copy(pe_hbm, pe_v)

        def seq_body(s, carry):
            base = (wid * seq_per_w + s) * SEQ_LEN
            pltpu.sync_copy(idx_hbm.at[pl.ds(base, _SPLIT_A)], idx_a)
            pltpu.sync_copy(
                idx_hbm.at[pl.ds(base + _SPLIT_A, _SPLIT_B)], idx_b)
            cp1 = pltpu.async_copy(
                table_hbm.at[idx_a], buf_v.at[pl.ds(0, _SPLIT_A)], sem)
            cp2 = pltpu.async_copy(
                table_hbm.at[idx_b], buf_v.at[pl.ds(_SPLIT_A, _SPLIT_B)], sem)
            cp1.wait()
            cp2.wait()

            def row_body(r, c):
                for j in range(D_MODEL // 16):
                    sl = pl.ds(j * 16, 16)
                    buf_v[r, sl] = buf_v[r, sl] * SCALE + pe_v[r, sl]
                return c

            lax.fori_loop(0, SEQ_LEN, row_body, 0, unroll=2)
            pltpu.sync_copy(buf_v, out_hbm.at[pl.ds(base, SEQ_LEN)])
            return carry

        lax.fori_loop(0, seq_per_w, seq_body, 0)

    return _emb_lookup


def kernel(x, table):
    b, s = x.shape
    idx = x.reshape(-1).astype(jnp.int32)
    pe = jnp.asarray(_PE_NP)
    out = _build_emb_lookup()(idx, table, pe)
    return out.reshape(b, s, D_MODEL)


# speedup vs baseline: 1.0909x; 1.0909x over previous
import functools

import numpy as np
import jax
import jax.numpy as jnp
from jax import lax
from jax.experimental import pallas as pl
from jax.experimental.pallas import tpu as pltpu
from jax.experimental.pallas import tpu_sc as plsc

D_MODEL = 64
SEQ_LEN = 200
BATCH = 1024
SCALE = np.float32(np.sqrt(D_MODEL))
NBUF = 3


def _positional_encoding(length, depth):
    half = depth / 2
    positions = np.arange(length)[:, np.newaxis]
    depths = np.arange(half)[np.newaxis, :] / half
    angle_rates = 1 / 10000 ** depths
    angle_rads = positions * angle_rates
    pe = np.concatenate([np.sin(angle_rads), np.cos(angle_rads)], axis=-1)
    return pe.astype(np.float32)


_PE_NP = _positional_encoding(SEQ_LEN, D_MODEL)

_SPLIT_A = 128
_SPLIT_B = SEQ_LEN - _SPLIT_A


@functools.cache
def _build_emb_lookup():
    info = plsc.get_sparse_core_info()
    nc, ns = info.num_cores, info.num_subcores
    nw = nc * ns
    seq_per_w = BATCH // nw
    mesh = plsc.VectorSubcoreMesh(core_axis_name="c", subcore_axis_name="s")

    @functools.partial(
        pl.kernel,
        mesh=mesh,
        out_type=jax.ShapeDtypeStruct((BATCH, SEQ_LEN, D_MODEL), jnp.float32),
        scratch_types=[
            pltpu.VMEM((seq_per_w * SEQ_LEN,), jnp.int32),
            pltpu.VMEM((SEQ_LEN, D_MODEL), jnp.float32),
            [pltpu.VMEM((SEQ_LEN, D_MODEL), jnp.float32)] * NBUF,
            [pltpu.SemaphoreType.DMA] * NBUF,
            [pltpu.SemaphoreType.DMA] * NBUF,
        ],
        compiler_params=pltpu.CompilerParams(use_tc_tiling_on_sc=False),
    )
    def _emb_lookup(idx_hbm, table_hbm, pe_hbm, out_hbm,
                    idx_v, pe_v, bufs, gsems, osems):
        wid = lax.axis_index("s") * nc + lax.axis_index("c")
        w_base = wid * seq_per_w

        pltpu.sync_copy(pe_hbm, pe_v)
        pltpu.sync_copy(idx_hbm.at[pl.ds(w_base * SEQ_LEN, seq_per_w * SEQ_LEN)],
                        idx_v)

        def fire_gather(s, b):
            base = s * SEQ_LEN
            c1 = pltpu.async_copy(
                table_hbm.at[idx_v.at[pl.ds(base, _SPLIT_A)]],
                bufs[b].at[pl.ds(0, _SPLIT_A)], gsems[b])
            c2 = pltpu.async_copy(
                table_hbm.at[idx_v.at[pl.ds(base + _SPLIT_A, _SPLIT_B)]],
                bufs[b].at[pl.ds(_SPLIT_A, _SPLIT_B)], gsems[b])
            return c1, c2

        pending_g = {}
        pending_o = {}
        for s in range(NBUF - 1):
            pending_g[s] = fire_gather(s, s)

        for s in range(seq_per_w):
            b = s % NBUF
            c1, c2 = pending_g.pop(s)
            c1.wait()
            c2.wait()
            buf = bufs[b]

            def row_body(r, c, buf=buf):
                for j in range(D_MODEL // 16):
                    sl = pl.ds(j * 16, 16)
                    buf[r, sl] = buf[r, sl] * SCALE + pe_v[r, sl]
                return c

            lax.fori_loop(0, SEQ_LEN, row_body, 0, unroll=2)

            pending_o[s] = pltpu.async_copy(
                buf, out_hbm.at[w_base + s], osems[b])

            nxt = s + NBUF - 1
            if nxt < seq_per_w:
                nb = nxt % NBUF
                if nxt - NBUF in pending_o:
                    pending_o.pop(nxt - NBUF).wait()
                pending_g[nxt] = fire_gather(nxt, nb)

        for s in sorted(pending_o):
            pending_o[s].wait()

    return _emb_lookup


def kernel(x, table):
    b, s = x.shape
    idx = x.reshape(-1).astype(jnp.int32)
    pe = jnp.asarray(_PE_NP)
    return _build_emb_lookup()(idx, table, pe)
